# Initial kernel scaffold; baseline (speedup 1.0000x reference)
import jax
import jax.numpy as jnp
from jax.experimental import pallas as pl


def kernel(z, batch, Wa1, ba1, Wa2, ba2, W1, b1, W2, b2, W3, b3):
    raise NotImplementedError("write your pallas kernel here")


# speedup vs baseline: 4.0743x; 4.0743x over previous
import functools

import jax
import jax.numpy as jnp
from jax import lax
from jax.experimental import pallas as pl
from jax.experimental.pallas import tpu as pltpu


def _att_kernel(z_ref, wa1_ref, ba1_ref, wa2_ref, a_ref, m_ref):
    j = pl.program_id(0)
    z = z_ref[...]
    h = lax.dot_general(z, wa1_ref[...], (((1,), (1,)), ((), ())),
                        preferred_element_type=jnp.float32)
    h = jnp.tanh(h + ba1_ref[...])
    a = lax.dot_general(h, wa2_ref[...], (((1,), (1,)), ((), ())),
                        preferred_element_type=jnp.float32)
    a_ref[...] = a

    bmax = jnp.max(a).reshape(1, 1)

    @pl.when(j == 0)
    def _():
        m_ref[...] = bmax

    @pl.when(j > 0)
    def _():
        m_ref[...] = jnp.maximum(m_ref[...], bmax)


def _seg_mlp_kernel(z_ref, a_ref, m_ref, b_ref,
                    w1_ref, b1_ref, w2_ref, b2_ref, w3_ref, b3_ref,
                    out_ref, acc_ref, zsum_ref, *, nblocks, num_segments):
    j = pl.program_id(0)

    @pl.when(j == 0)
    def _():
        acc_ref[...] = jnp.zeros_like(acc_ref)
        zsum_ref[0, 0] = 0.0

    w = jnp.exp(a_ref[...] - m_ref[...])
    zsum_ref[0, 0] += jnp.sum(w)

    ids = b_ref[0, 0, :]
    seg = lax.broadcasted_iota(jnp.int32, (num_segments, ids.shape[0]), 0)
    onehot = (seg == ids[None, :]).astype(jnp.float32)
    wz = z_ref[...] * w
    acc_ref[...] += jnp.dot(onehot, wz, preferred_element_type=jnp.float32)

    @pl.when(j == nblocks - 1)
    def _():
        gr = acc_ref[...] / zsum_ref[0, 0]
        h1 = jnp.maximum(
            lax.dot_general(gr, w1_ref[...], (((1,), (1,)), ((), ())),
                            preferred_element_type=jnp.float32) + b1_ref[...],
            0.0)
        h2 = jnp.maximum(
            lax.dot_general(h1, w2_ref[...], (((1,), (1,)), ((), ())),
                            preferred_element_type=jnp.float32) + b2_ref[...],
            0.0)
        out_ref[...] = lax.dot_general(
            h2, w3_ref[...], (((1,), (1,)), ((), ())),
            preferred_element_type=jnp.float32) + b3_ref[...]


def kernel(z, batch, Wa1, ba1, Wa2, ba2, W1, b1, W2, b2, W3, b3):
    n, d = z.shape
    h = Wa1.shape[0]
    g = 512
    c = W3.shape[0]
    hh = W2.shape[0]

    blk = 2000
    if n % blk != 0:
        blk = n
    nb = n // blk

    batch32 = batch.astype(jnp.int32).reshape(nb, 1, blk)
    ba1_2 = ba1.reshape(1, h)

    a, m = pl.pallas_call(
        _att_kernel,
        grid=(nb,),
        in_specs=[
            pl.BlockSpec((blk, d), lambda j: (j, 0)),
            pl.BlockSpec((h, d), lambda j: (0, 0)),
            pl.BlockSpec((1, h), lambda j: (0, 0)),
            pl.BlockSpec((1, h), lambda j: (0, 0)),
        ],
        out_specs=[
            pl.BlockSpec((blk, 1), lambda j: (j, 0)),
            pl.BlockSpec((1, 1), lambda j: (0, 0)),
        ],
        out_shape=[
            jax.ShapeDtypeStruct((n, 1), jnp.float32),
            jax.ShapeDtypeStruct((1, 1), jnp.float32),
        ],
    )(z, Wa1, ba1_2, Wa2)

    out = pl.pallas_call(
        functools.partial(_seg_mlp_kernel, nblocks=nb, num_segments=g),
        grid=(nb,),
        in_specs=[
            pl.BlockSpec((blk, d), lambda j: (j, 0)),
            pl.BlockSpec((blk, 1), lambda j: (j, 0)),
            pl.BlockSpec((1, 1), lambda j: (0, 0)),
            pl.BlockSpec((1, 1, blk), lambda j: (j, 0, 0)),
            pl.BlockSpec((h, d), lambda j: (0, 0)),
            pl.BlockSpec((1, h), lambda j: (0, 0)),
            pl.BlockSpec((hh, h), lambda j: (0, 0)),
            pl.BlockSpec((1, hh), lambda j: (0, 0)),
            pl.BlockSpec((c, hh), lambda j: (0, 0)),
            pl.BlockSpec((1, c), lambda j: (0, 0)),
        ],
        out_specs=pl.BlockSpec((g, c), lambda j: (0, 0)),
        out_shape=jax.ShapeDtypeStruct((g, c), jnp.float32),
        scratch_shapes=[
            pltpu.VMEM((g, d), jnp.float32),
            pltpu.SMEM((1, 1), jnp.float32),
        ],
    )(z, a, m, batch32, W1, b1.reshape(1, h), W2, b2.reshape(1, hh),
      W3, b3.reshape(1, c))

    return out
